# Initial kernel scaffold; baseline (speedup 1.0000x reference)
import jax
import jax.numpy as jnp
from jax.experimental import pallas as pl


def kernel(input, lut, W_ih, W_hh, b_ih, b_hh, phrase_tokens, phrase_lens):
    raise NotImplementedError("write your pallas kernel here")


# speedup vs baseline: 9.7203x; 9.7203x over previous
import functools

import jax
import jax.numpy as jnp
from jax import lax
from jax.experimental import pallas as pl
from jax.experimental.pallas import tpu as pltpu, tpu_sc as plsc

V = 100000
P = 1024
H = 128
PAD = 1
L, B = 200, 1024
MAXP = 5

NC, NS = 2, 16
NW = NC * NS


def _sc_gather(n_rows, chunk):
    b_per_w = n_rows // NW
    assert n_rows % NW == 0 and b_per_w % chunk == 0 and chunk % 8 == 0
    n_chunks = b_per_w // chunk

    def body(table_hbm, idx_hbm, out_hbm, idx_v, rows_v, sem):
        wid = lax.axis_index("s") * NC + lax.axis_index("c")
        base = wid * b_per_w
        pltpu.sync_copy(idx_hbm.at[pl.ds(base, b_per_w)], idx_v)
        for c in range(n_chunks):
            pltpu.async_copy(
                table_hbm.at[idx_v.at[pl.ds(c * chunk, chunk)]],
                rows_v, sem).wait()
            pltpu.sync_copy(rows_v, out_hbm.at[pl.ds(base + c * chunk, chunk)])

    def run(table, idx):
        mesh = plsc.VectorSubcoreMesh(core_axis_name="c", subcore_axis_name="s",
                                      num_cores=NC, num_subcores=NS)
        return pl.kernel(
            body,
            out_type=jax.ShapeDtypeStruct((n_rows, H), jnp.float32),
            mesh=mesh,
            scratch_types=[
                pltpu.VMEM((b_per_w,), jnp.int32),
                pltpu.VMEM((chunk, H), jnp.float32),
                pltpu.SemaphoreType.DMA,
            ],
        )(table, idx)

    return run


_gather_small = _sc_gather(P * MAXP, P * MAXP // NW)
_gather_big = _sc_gather(L * B, 400)


def _lstm_body(emb_ref, wih_ref, whh_ref, bih_ref, bhh_ref, lens_ref, out_ref):
    wih = wih_ref[...]
    whh = whh_ref[...]
    b = bih_ref[...] + bhh_ref[...]
    lens = lens_ref[...]
    h = jnp.zeros((P, H), jnp.float32)
    c = jnp.zeros((P, H), jnp.float32)
    for t in range(MAXP):
        x = emb_ref[t]
        gates = (
            lax.dot_general(x, wih, (((1,), (1,)), ((), ())),
                            precision=lax.Precision.HIGHEST)
            + lax.dot_general(h, whh, (((1,), (1,)), ((), ())),
                              precision=lax.Precision.HIGHEST)
            + b)
        i_g = jax.nn.sigmoid(gates[:, 0 * H:1 * H])
        f_g = jax.nn.sigmoid(gates[:, 1 * H:2 * H])
        g_g = jnp.tanh(gates[:, 2 * H:3 * H])
        o_g = jax.nn.sigmoid(gates[:, 3 * H:4 * H])
        c_new = f_g * c + i_g * g_g
        h_new = o_g * jnp.tanh(c_new)
        valid = lens > t
        h = jnp.where(valid, h_new, h)
        c = jnp.where(valid, c_new, c)
    out_ref[...] = h


def _lstm(emb, W_ih, W_hh, b_ih, b_hh, lens):
    return pl.pallas_call(
        _lstm_body,
        out_shape=jax.ShapeDtypeStruct((P, H), jnp.float32),
    )(emb, W_ih, W_hh, b_ih.reshape(1, 4 * H), b_hh.reshape(1, 4 * H),
      lens.reshape(P, 1))


def kernel(input, lut, W_ih, W_hh, b_ih, b_hh, phrase_tokens, phrase_lens):
    inp = input[:, :, 0].astype(jnp.int32).reshape(L * B)
    pt = phrase_tokens.astype(jnp.int32).T.reshape(MAXP * P)
    lens = phrase_lens.astype(jnp.int32)
    emb = _gather_small(lut, pt).reshape(MAXP, P, H)
    phrases_h = _lstm(emb, W_ih, W_hh, b_ih, b_hh, lens)
    table = jnp.concatenate([lut, phrases_h], axis=0)
    out = _gather_big(table, inp)
    return out.reshape(L, B, H)


# speedup vs baseline: 10.3789x; 1.0678x over previous
import functools

import jax
import jax.numpy as jnp
from jax import lax
from jax.experimental import pallas as pl
from jax.experimental.pallas import tpu as pltpu, tpu_sc as plsc

V = 100000
P = 1024
H = 128
PAD = 1
L, B = 200, 1024
MAXP = 5

NC, NS = 2, 16
NW = NC * NS


def _sc_gather(n_rows, chunk):
    b_per_w = n_rows // NW
    assert n_rows % NW == 0 and b_per_w % chunk == 0 and chunk % 8 == 0
    n_chunks = b_per_w // chunk

    def body(table_hbm, idx_hbm, out_hbm, idx_v, rows0, rows1, gs0, gs1,
             ws0, ws1):
        wid = lax.axis_index("s") * NC + lax.axis_index("c")
        base = wid * b_per_w
        pltpu.sync_copy(idx_hbm.at[pl.ds(base, b_per_w)], idx_v)
        bufs, gsems, wsems = (rows0, rows1), (gs0, gs1), (ws0, ws1)

        def gather(c):
            return pltpu.async_copy(
                table_hbm.at[idx_v.at[pl.ds(c * chunk, chunk)]],
                bufs[c % 2], gsems[c % 2])

        def put(c):
            return pltpu.async_copy(
                bufs[c % 2], out_hbm.at[pl.ds(base + c * chunk, chunk)],
                wsems[c % 2])

        g = {0: gather(0)}
        w = {}
        for c in range(n_chunks):
            if c + 1 < n_chunks:
                if c >= 1:
                    w.pop(c - 1).wait()
                g[c + 1] = gather(c + 1)
            g.pop(c).wait()
            w[c] = put(c)
        for c in sorted(w):
            w.pop(c).wait()

    def run(table, idx):
        mesh = plsc.VectorSubcoreMesh(core_axis_name="c", subcore_axis_name="s",
                                      num_cores=NC, num_subcores=NS)
        return pl.kernel(
            body,
            out_type=jax.ShapeDtypeStruct((n_rows, H), jnp.float32),
            mesh=mesh,
            scratch_types=[
                pltpu.VMEM((b_per_w,), jnp.int32),
                pltpu.VMEM((chunk, H), jnp.float32),
                pltpu.VMEM((chunk, H), jnp.float32),
                pltpu.SemaphoreType.DMA,
                pltpu.SemaphoreType.DMA,
                pltpu.SemaphoreType.DMA,
                pltpu.SemaphoreType.DMA,
            ],
        )(table, idx)

    return run


_gather_small = _sc_gather(P * MAXP, P * MAXP // NW)
_gather_big = _sc_gather(L * B, 400)


def _lstm_body(emb_ref, wih_ref, whh_ref, bih_ref, bhh_ref, lens_ref, out_ref):
    wih = wih_ref[...]
    whh = whh_ref[...]
    b = bih_ref[...] + bhh_ref[...]
    lens = lens_ref[...]
    h = jnp.zeros((P, H), jnp.float32)
    c = jnp.zeros((P, H), jnp.float32)
    for t in range(MAXP):
        x = emb_ref[t]
        gates = (
            lax.dot_general(x, wih, (((1,), (1,)), ((), ())),
                            precision=lax.Precision.HIGHEST)
            + lax.dot_general(h, whh, (((1,), (1,)), ((), ())),
                              precision=lax.Precision.HIGHEST)
            + b)
        i_g = jax.nn.sigmoid(gates[:, 0 * H:1 * H])
        f_g = jax.nn.sigmoid(gates[:, 1 * H:2 * H])
        g_g = jnp.tanh(gates[:, 2 * H:3 * H])
        o_g = jax.nn.sigmoid(gates[:, 3 * H:4 * H])
        c_new = f_g * c + i_g * g_g
        h_new = o_g * jnp.tanh(c_new)
        valid = lens > t
        h = jnp.where(valid, h_new, h)
        c = jnp.where(valid, c_new, c)
    out_ref[...] = h


def _lstm(emb, W_ih, W_hh, b_ih, b_hh, lens):
    return pl.pallas_call(
        _lstm_body,
        out_shape=jax.ShapeDtypeStruct((P, H), jnp.float32),
    )(emb, W_ih, W_hh, b_ih.reshape(1, 4 * H), b_hh.reshape(1, 4 * H),
      lens.reshape(P, 1))


def kernel(input, lut, W_ih, W_hh, b_ih, b_hh, phrase_tokens, phrase_lens):
    inp = input[:, :, 0].astype(jnp.int32).reshape(L * B)
    pt = phrase_tokens.astype(jnp.int32).T.reshape(MAXP * P)
    lens = phrase_lens.astype(jnp.int32)
    emb = _gather_small(lut, pt).reshape(MAXP, P, H)
    phrases_h = _lstm(emb, W_ih, W_hh, b_ih, b_hh, lens)
    table = jnp.concatenate([lut, phrases_h], axis=0)
    out = _gather_big(table, inp)
    return out.reshape(L, B, H)
